# baseline (speedup 1.0000x reference)
import dataclasses

import jax
import jax.numpy as jnp
from jax import lax
from jax.experimental import pallas as pl
from jax.experimental.pallas import tpu as pltpu
from jax.experimental.pallas import tpu_sc as plsc

_BINS = 64
_N = 16777216
_LANES = 128
_ROWS = _N // _LANES
_BLK_ROWS = 4096
_GRID = _ROWS // _BLK_ROWS

_G = 16
_FBINS = _BINS * _G
_NC, _NS, _L = 2, 16, 16
_NW = _NC * _NS

_TC_ROWS = 16384
_SC_ROWS = _ROWS - _TC_ROWS
_TC_BLOCKS = _TC_ROWS // _BLK_ROWS
_SC_BLOCKS = _SC_ROWS // _BLK_ROWS
_SC_N = _SC_ROWS * _LANES
_TILE_N = _SC_N // _NW
_CHUNK = 28672
_NCHUNK = _TILE_N // _CHUNK
_UNROLL = 8


def _minmax_body(o_ref, t_ref, ms_ref, mn_ref, mx_ref, mn_sm, mx_sm):
    i = pl.program_id(0)
    bmn = jnp.minimum(jnp.min(o_ref[...]), jnp.min(t_ref[...]))
    bmx = jnp.maximum(jnp.max(o_ref[...]), jnp.max(t_ref[...]))

    @pl.when(i == 0)
    def _():
        mn_sm[0] = bmn
        mx_sm[0] = bmx

    @pl.when(i != 0)
    def _():
        mn_sm[0] = jnp.minimum(mn_sm[0], bmn)
        mx_sm[0] = jnp.maximum(mx_sm[0], bmx)

    @pl.when(i == _GRID - 1)
    def _():
        mn = mn_sm[0]
        mx = mx_sm[0]
        fscale = _FBINS / (mx - mn)
        ms_ref[0:1, :] = jnp.full((1, _LANES), mn, jnp.float32)
        ms_ref[1:2, :] = jnp.full((1, _LANES), fscale, jnp.float32)
        mn_ref[0, 0] = mn
        mx_ref[0, 0] = mx


def _sc_hist_body(o_hbm, t_hbm, ms_hbm, out_hbm, mn_v, sc_v, ob0, ob1, tb0,
                  tb1, fo, ft, co, ct, sems):
    cid = lax.axis_index("c")
    sid = lax.axis_index("s")
    wid = sid * _NC + cid
    base = wid * _TILE_N

    pltpu.sync_copy(ms_hbm.at[0], mn_v)
    pltpu.sync_copy(ms_hbm.at[1], sc_v)
    mn = mn_v[pl.ds(0, _L)]
    sc = sc_v[pl.ds(0, _L)]

    zeros = jnp.zeros((_L,), jnp.float32)
    for h in (fo, ft):
        for k in range(_FBINS // _L):
            h[pl.ds(k * _L, _L)] = zeros

    obufs = (ob0, ob1)
    tbufs = (tb0, tb1)

    def start(slot, c):
        off = base + c * _CHUNK
        pltpu.async_copy(o_hbm.at[pl.ds(off, _CHUNK)], obufs[slot],
                         sems.at[0, slot])
        pltpu.async_copy(t_hbm.at[pl.ds(off, _CHUNK)], tbufs[slot],
                         sems.at[1, slot])

    def wait(slot):
        pltpu.make_async_copy(o_hbm.at[pl.ds(0, _CHUNK)], obufs[slot],
                              sems.at[0, slot]).wait()
        pltpu.make_async_copy(t_hbm.at[pl.ds(0, _CHUNK)], tbufs[slot],
                              sems.at[1, slot]).wait()

    start(0, 0)
    start(1, 1)

    ones = jnp.full((_L,), 1.0, jnp.float32)
    topf = jnp.full((_L,), float(_FBINS - 1), jnp.float32)

    def binvec(x):
        t = jnp.minimum((x - mn) * sc, topf)
        return t.astype(jnp.int32)

    def process(obuf, tbuf):
        @plsc.parallel_loop(0, _CHUNK, step=_L, unroll=_UNROLL)
        def _(j):
            plsc.addupdate_scatter(fo, [binvec(obuf[pl.ds(j, _L)])], ones)
            plsc.addupdate_scatter(ft, [binvec(tbuf[pl.ds(j, _L)])], ones)

    @pl.loop(0, _NCHUNK, step=2)
    def _(c):
        for b in range(2):
            wait(b)
            process(obufs[b], tbufs[b])

            @pl.when(c + (b + 2) < _NCHUNK)
            def _():
                start(b, c + (b + 2))

    glanes = lax.iota(jnp.int32, _L) * _G
    for k in range(_BINS // _L):
        acc_o = jnp.zeros((_L,), jnp.float32)
        acc_t = jnp.zeros((_L,), jnp.float32)
        for j in range(_G):
            gidx = glanes + (k * _L * _G + j)
            acc_o = acc_o + plsc.load_gather(fo, [gidx])
            acc_t = acc_t + plsc.load_gather(ft, [gidx])
        co[pl.ds(k * _L, _L)] = acc_o
        ct[pl.ds(k * _L, _L)] = acc_t

    pltpu.sync_copy(co, out_hbm.at[wid, pl.ds(0, _BINS)])
    pltpu.sync_copy(ct, out_hbm.at[wid, pl.ds(_BINS, _BINS)])


def _tc_hist_body(mn_ref, mx_ref, o_ref, t_ref, acc_ref):
    i = pl.program_id(0)

    @pl.when(i == 0)
    def _():
        acc_ref[...] = jnp.zeros_like(acc_ref)

    mn = mn_ref[0, 0]
    mx = mx_ref[0, 0]
    scale = _BINS / (mx - mn)
    idx_o = jnp.floor((o_ref[...] - mn) * scale).astype(jnp.int32)
    idx_o = jnp.clip(idx_o, 0, _BINS - 1)
    idx_t = jnp.floor((t_ref[...] - mn) * scale).astype(jnp.int32)
    idx_t = jnp.clip(idx_t, 0, _BINS - 1)

    for b in range(_BINS):
        so = jnp.sum((idx_o == b).astype(jnp.float32), axis=0, keepdims=True)
        st = jnp.sum((idx_t == b).astype(jnp.float32), axis=0, keepdims=True)
        acc_ref[b:b + 1, :] += so
        acc_ref[_BINS + b:_BINS + b + 1, :] += st


def _loss_body(ho_ref, ht_ref, ao_ref, at_ref, loss_ref):
    d = (jnp.sum(ho_ref[...], axis=0) + jnp.sum(ao_ref[...], axis=0)
         - jnp.sum(ht_ref[...], axis=0) - jnp.sum(at_ref[...], axis=0))
    loss_ref[0, 0] = jnp.mean(jnp.abs(d))


def kernel(output, target):
    o2 = output.reshape(_ROWS, _LANES)
    t2 = target.reshape(_ROWS, _LANES)

    ms, mn11, mx11 = pl.pallas_call(
        _minmax_body,
        grid=(_GRID,),
        in_specs=[
            pl.BlockSpec((_BLK_ROWS, _LANES), lambda i: (i, 0)),
            pl.BlockSpec((_BLK_ROWS, _LANES), lambda i: (i, 0)),
        ],
        out_specs=[
            pl.BlockSpec((2, _LANES), lambda i: (0, 0)),
            pl.BlockSpec((1, 1), lambda i: (0, 0), memory_space=pltpu.SMEM),
            pl.BlockSpec((1, 1), lambda i: (0, 0), memory_space=pltpu.SMEM),
        ],
        out_shape=[
            jax.ShapeDtypeStruct((2, _LANES), jnp.float32),
            jax.ShapeDtypeStruct((1, 1), jnp.float32),
            jax.ShapeDtypeStruct((1, 1), jnp.float32),
        ],
        scratch_shapes=[
            pltpu.SMEM((1,), jnp.float32),
            pltpu.SMEM((1,), jnp.float32),
        ],
        compiler_params=pltpu.CompilerParams(
            dimension_semantics=("arbitrary",),
        ),
    )(o2, t2)

    sc_params = pltpu.CompilerParams()
    if "needs_layout_passes" in pltpu.CompilerParams.__dataclass_fields__:
        sc_params = dataclasses.replace(sc_params, needs_layout_passes=False)

    sc_hist = pl.kernel(
        _sc_hist_body,
        compiler_params=sc_params,
        out_type=jax.ShapeDtypeStruct((_NW, 2 * _BINS), jnp.float32),
        mesh=plsc.VectorSubcoreMesh(core_axis_name="c", subcore_axis_name="s",
                                    num_cores=_NC, num_subcores=_NS),
        scratch_types=[
            pltpu.VMEM((_LANES,), jnp.float32),
            pltpu.VMEM((_LANES,), jnp.float32),
            pltpu.VMEM((_CHUNK,), jnp.float32),
            pltpu.VMEM((_CHUNK,), jnp.float32),
            pltpu.VMEM((_CHUNK,), jnp.float32),
            pltpu.VMEM((_CHUNK,), jnp.float32),
            pltpu.VMEM((_FBINS,), jnp.float32),
            pltpu.VMEM((_FBINS,), jnp.float32),
            pltpu.VMEM((_BINS,), jnp.float32),
            pltpu.VMEM((_BINS,), jnp.float32),
            pltpu.SemaphoreType.DMA((2, 2)),
        ],
    )
    hp = sc_hist(output, target, ms)

    tc_acc = pl.pallas_call(
        _tc_hist_body,
        grid=(_TC_BLOCKS,),
        in_specs=[
            pl.BlockSpec((1, 1), lambda i: (0, 0), memory_space=pltpu.SMEM),
            pl.BlockSpec((1, 1), lambda i: (0, 0), memory_space=pltpu.SMEM),
            pl.BlockSpec((_BLK_ROWS, _LANES), lambda i: (_SC_BLOCKS + i, 0)),
            pl.BlockSpec((_BLK_ROWS, _LANES), lambda i: (_SC_BLOCKS + i, 0)),
        ],
        out_specs=pl.BlockSpec((2 * _BINS, _LANES), lambda i: (0, 0)),
        out_shape=jax.ShapeDtypeStruct((2 * _BINS, _LANES), jnp.float32),
        compiler_params=pltpu.CompilerParams(
            dimension_semantics=("arbitrary",),
        ),
    )(mn11, mx11, o2, t2)

    acc_t = tc_acc.T

    loss = pl.pallas_call(
        _loss_body,
        out_specs=pl.BlockSpec(memory_space=pltpu.SMEM),
        out_shape=jax.ShapeDtypeStruct((1, 1), jnp.float32),
    )(hp[:, :_BINS], hp[:, _BINS:], acc_t[:, :_BINS], acc_t[:, _BINS:])

    return loss[0, 0]


# speedup vs baseline: 1.0072x; 1.0072x over previous
import dataclasses

import jax
import jax.numpy as jnp
from jax import lax
from jax.experimental import pallas as pl
from jax.experimental.pallas import tpu as pltpu
from jax.experimental.pallas import tpu_sc as plsc

_BINS = 64
_N = 16777216
_LANES = 128
_ROWS = _N // _LANES
_BLK_ROWS = 4096
_GRID = _ROWS // _BLK_ROWS

_G = 16
_FBINS = _BINS * _G
_NC, _NS, _L = 2, 16, 16
_NW = _NC * _NS

_TC_ROWS = 16384
_SC_ROWS = _ROWS - _TC_ROWS
_TC_BLOCKS = _TC_ROWS // _BLK_ROWS
_SC_BLOCKS = _SC_ROWS // _BLK_ROWS
_SC_N = _SC_ROWS * _LANES
_TILE_N = _SC_N // _NW
_CHUNK = 16384
_NCHUNK = _TILE_N // _CHUNK
_UNROLL = 8


def _minmax_body(o_ref, t_ref, ms_ref, mn_ref, mx_ref, mn_sm, mx_sm):
    i = pl.program_id(0)
    bmn = jnp.minimum(jnp.min(o_ref[...]), jnp.min(t_ref[...]))
    bmx = jnp.maximum(jnp.max(o_ref[...]), jnp.max(t_ref[...]))

    @pl.when(i == 0)
    def _():
        mn_sm[0] = bmn
        mx_sm[0] = bmx

    @pl.when(i != 0)
    def _():
        mn_sm[0] = jnp.minimum(mn_sm[0], bmn)
        mx_sm[0] = jnp.maximum(mx_sm[0], bmx)

    @pl.when(i == _GRID - 1)
    def _():
        mn = mn_sm[0]
        mx = mx_sm[0]
        fscale = _FBINS / (mx - mn)
        ms_ref[0:1, :] = jnp.full((1, _LANES), mn, jnp.float32)
        ms_ref[1:2, :] = jnp.full((1, _LANES), fscale, jnp.float32)
        mn_ref[0, 0] = mn
        mx_ref[0, 0] = mx


def _sc_hist_body(o_hbm, t_hbm, ms_hbm, out_hbm, mn_v, sc_v, ob0, ob1, tb0,
                  tb1, fo, ft, co, ct, sems):
    cid = lax.axis_index("c")
    sid = lax.axis_index("s")
    wid = sid * _NC + cid
    base = wid * _TILE_N

    pltpu.sync_copy(ms_hbm.at[0], mn_v)
    pltpu.sync_copy(ms_hbm.at[1], sc_v)
    mn = mn_v[pl.ds(0, _L)]
    sc = sc_v[pl.ds(0, _L)]

    zeros = jnp.zeros((_L,), jnp.float32)
    for h in (fo, ft):
        for k in range(_FBINS // _L):
            h[pl.ds(k * _L, _L)] = zeros

    obufs = (ob0, ob1)
    tbufs = (tb0, tb1)

    def start(slot, c):
        off = base + c * _CHUNK
        pltpu.async_copy(o_hbm.at[pl.ds(off, _CHUNK)], obufs[slot],
                         sems.at[0, slot])
        pltpu.async_copy(t_hbm.at[pl.ds(off, _CHUNK)], tbufs[slot],
                         sems.at[1, slot])

    def wait(slot):
        pltpu.make_async_copy(o_hbm.at[pl.ds(0, _CHUNK)], obufs[slot],
                              sems.at[0, slot]).wait()
        pltpu.make_async_copy(t_hbm.at[pl.ds(0, _CHUNK)], tbufs[slot],
                              sems.at[1, slot]).wait()

    start(0, 0)
    start(1, 1)

    ones = jnp.full((_L,), 1.0, jnp.float32)
    topf = jnp.full((_L,), float(_FBINS - 1), jnp.float32)

    def binvec(x):
        t = jnp.minimum((x - mn) * sc, topf)
        return t.astype(jnp.int32)

    def process(obuf, tbuf):
        @plsc.parallel_loop(0, _CHUNK, step=_L, unroll=_UNROLL)
        def _(j):
            plsc.addupdate_scatter(fo, [binvec(obuf[pl.ds(j, _L)])], ones)
            plsc.addupdate_scatter(ft, [binvec(tbuf[pl.ds(j, _L)])], ones)

    @pl.loop(0, _NCHUNK, step=2)
    def _(c):
        for b in range(2):
            wait(b)
            process(obufs[b], tbufs[b])

            @pl.when(c + (b + 2) < _NCHUNK)
            def _():
                start(b, c + (b + 2))

    glanes = lax.iota(jnp.int32, _L) * _G
    for k in range(_BINS // _L):
        acc_o = jnp.zeros((_L,), jnp.float32)
        acc_t = jnp.zeros((_L,), jnp.float32)
        for j in range(_G):
            gidx = glanes + (k * _L * _G + j)
            acc_o = acc_o + plsc.load_gather(fo, [gidx])
            acc_t = acc_t + plsc.load_gather(ft, [gidx])
        co[pl.ds(k * _L, _L)] = acc_o
        ct[pl.ds(k * _L, _L)] = acc_t

    pltpu.sync_copy(co, out_hbm.at[wid, pl.ds(0, _BINS)])
    pltpu.sync_copy(ct, out_hbm.at[wid, pl.ds(_BINS, _BINS)])


def _tc_hist_body(mn_ref, mx_ref, o_ref, t_ref, acc_ref):
    i = pl.program_id(0)

    @pl.when(i == 0)
    def _():
        acc_ref[...] = jnp.zeros_like(acc_ref)

    mn = mn_ref[0, 0]
    mx = mx_ref[0, 0]
    scale = _BINS / (mx - mn)
    idx_o = jnp.floor((o_ref[...] - mn) * scale).astype(jnp.int32)
    idx_o = jnp.clip(idx_o, 0, _BINS - 1)
    idx_t = jnp.floor((t_ref[...] - mn) * scale).astype(jnp.int32)
    idx_t = jnp.clip(idx_t, 0, _BINS - 1)

    for b in range(_BINS):
        so = jnp.sum((idx_o == b).astype(jnp.float32), axis=0, keepdims=True)
        st = jnp.sum((idx_t == b).astype(jnp.float32), axis=0, keepdims=True)
        acc_ref[b:b + 1, :] += so
        acc_ref[_BINS + b:_BINS + b + 1, :] += st


def _loss_body(ho_ref, ht_ref, ao_ref, at_ref, loss_ref):
    d = (jnp.sum(ho_ref[...], axis=0) + jnp.sum(ao_ref[...], axis=0)
         - jnp.sum(ht_ref[...], axis=0) - jnp.sum(at_ref[...], axis=0))
    loss_ref[0, 0] = jnp.mean(jnp.abs(d))


def kernel(output, target):
    o2 = output.reshape(_ROWS, _LANES)
    t2 = target.reshape(_ROWS, _LANES)

    ms, mn11, mx11 = pl.pallas_call(
        _minmax_body,
        grid=(_GRID,),
        in_specs=[
            pl.BlockSpec((_BLK_ROWS, _LANES), lambda i: (i, 0)),
            pl.BlockSpec((_BLK_ROWS, _LANES), lambda i: (i, 0)),
        ],
        out_specs=[
            pl.BlockSpec((2, _LANES), lambda i: (0, 0)),
            pl.BlockSpec((1, 1), lambda i: (0, 0), memory_space=pltpu.SMEM),
            pl.BlockSpec((1, 1), lambda i: (0, 0), memory_space=pltpu.SMEM),
        ],
        out_shape=[
            jax.ShapeDtypeStruct((2, _LANES), jnp.float32),
            jax.ShapeDtypeStruct((1, 1), jnp.float32),
            jax.ShapeDtypeStruct((1, 1), jnp.float32),
        ],
        scratch_shapes=[
            pltpu.SMEM((1,), jnp.float32),
            pltpu.SMEM((1,), jnp.float32),
        ],
        compiler_params=pltpu.CompilerParams(
            dimension_semantics=("arbitrary",),
        ),
    )(o2, t2)

    sc_params = pltpu.CompilerParams()
    if "needs_layout_passes" in pltpu.CompilerParams.__dataclass_fields__:
        sc_params = dataclasses.replace(sc_params, needs_layout_passes=False)

    sc_hist = pl.kernel(
        _sc_hist_body,
        compiler_params=sc_params,
        out_type=jax.ShapeDtypeStruct((_NW, 2 * _BINS), jnp.float32),
        mesh=plsc.VectorSubcoreMesh(core_axis_name="c", subcore_axis_name="s",
                                    num_cores=_NC, num_subcores=_NS),
        scratch_types=[
            pltpu.VMEM((_LANES,), jnp.float32),
            pltpu.VMEM((_LANES,), jnp.float32),
            pltpu.VMEM((_CHUNK,), jnp.float32),
            pltpu.VMEM((_CHUNK,), jnp.float32),
            pltpu.VMEM((_CHUNK,), jnp.float32),
            pltpu.VMEM((_CHUNK,), jnp.float32),
            pltpu.VMEM((_FBINS,), jnp.float32),
            pltpu.VMEM((_FBINS,), jnp.float32),
            pltpu.VMEM((_BINS,), jnp.float32),
            pltpu.VMEM((_BINS,), jnp.float32),
            pltpu.SemaphoreType.DMA((2, 2)),
        ],
    )
    hp = sc_hist(output, target, ms)

    tc_acc = pl.pallas_call(
        _tc_hist_body,
        grid=(_TC_BLOCKS,),
        in_specs=[
            pl.BlockSpec((1, 1), lambda i: (0, 0), memory_space=pltpu.SMEM),
            pl.BlockSpec((1, 1), lambda i: (0, 0), memory_space=pltpu.SMEM),
            pl.BlockSpec((_BLK_ROWS, _LANES), lambda i: (_SC_BLOCKS + i, 0)),
            pl.BlockSpec((_BLK_ROWS, _LANES), lambda i: (_SC_BLOCKS + i, 0)),
        ],
        out_specs=pl.BlockSpec((2 * _BINS, _LANES), lambda i: (0, 0)),
        out_shape=jax.ShapeDtypeStruct((2 * _BINS, _LANES), jnp.float32),
        compiler_params=pltpu.CompilerParams(
            dimension_semantics=("arbitrary",),
        ),
    )(mn11, mx11, o2, t2)

    acc_t = tc_acc.T

    loss = pl.pallas_call(
        _loss_body,
        out_specs=pl.BlockSpec(memory_space=pltpu.SMEM),
        out_shape=jax.ShapeDtypeStruct((1, 1), jnp.float32),
    )(hp[:, :_BINS], hp[:, _BINS:], acc_t[:, :_BINS], acc_t[:, _BINS:])

    return loss[0, 0]
